# baseline (speedup 1.0000x reference)
import functools

import jax
import jax.numpy as jnp
from jax import lax
from jax.experimental import pallas as pl
from jax.experimental.pallas import tpu as pltpu
from jax.experimental.pallas import tpu_sc as plsc

CHUNK = 800


def _make_embed(n_rows: int, dim: int):
    info = plsc.get_sparse_core_info()
    nc, ns = info.num_cores, info.num_subcores
    nw = nc * ns
    per_w = n_rows // nw
    chunk = CHUNK
    n_ch = per_w // chunk
    n_pairs = n_ch // 2

    mesh = plsc.VectorSubcoreMesh(core_axis_name="c", subcore_axis_name="s")

    @functools.partial(
        pl.kernel,
        mesh=mesh,
        out_type=jax.ShapeDtypeStruct((n_rows, dim), jnp.float32),
        scratch_types=[
            pltpu.VMEM((chunk,), jnp.int32),
            pltpu.VMEM((chunk,), jnp.int32),
            pltpu.VMEM((chunk, dim), jnp.float32),
            pltpu.VMEM((chunk, dim), jnp.float32),
            pltpu.SemaphoreType.DMA,
            pltpu.SemaphoreType.DMA,
            pltpu.SemaphoreType.DMA,
            pltpu.SemaphoreType.DMA,
        ],
        compiler_params=pltpu.CompilerParams(use_tc_tiling_on_sc=False),
    )
    def embed(idx_hbm, table_hbm, out_hbm, i0v, i1v, r0v, r1v, g0, g1, w0, w1):
        wid = lax.axis_index("s") * nc + lax.axis_index("c")
        base = wid * per_w
        idx_v = (i0v, i1v)
        rows_v = (r0v, r1v)
        gsem = (g0, g1)
        wsem = (w0, w1)

        def gather(i_dyn, b):
            off = base + i_dyn * chunk
            pltpu.sync_copy(idx_hbm.at[pl.ds(off, chunk)], idx_v[b])
            pltpu.async_copy(table_hbm.at[idx_v[b]], rows_v[b], gsem[b])

        def wait_gather(b):
            pltpu.make_async_copy(
                table_hbm.at[idx_v[b]], rows_v[b], gsem[b]
            ).wait()

        def writeback(i_dyn, b):
            off = base + i_dyn * chunk
            pltpu.async_copy(rows_v[b], out_hbm.at[pl.ds(off, chunk)], wsem[b])

        def wait_writeback(b):
            pltpu.make_async_copy(
                rows_v[b], out_hbm.at[pl.ds(base, chunk)], wsem[b]
            ).wait()

        gather(0, 0)

        def body(j, carry):
            i0 = 2 * j
            i1 = i0 + 1

            @pl.when(j > 0)
            def _():
                wait_writeback(1)

            gather(i1, 1)
            wait_gather(0)
            writeback(i0, 0)

            @pl.when(j + 1 < n_pairs)
            def _():
                wait_writeback(0)
                gather(i0 + 2, 0)

            wait_gather(1)
            writeback(i1, 1)
            return carry

        lax.fori_loop(0, n_pairs, body, 0)
        wait_writeback(0)
        wait_writeback(1)

    return embed


def kernel(event_activities, activity_embedding):
    t, b = event_activities.shape
    vocab, dim = activity_embedding.shape
    n_rows = t * b
    idx_flat = event_activities.reshape(n_rows).astype(jnp.int32)
    out = _make_embed(n_rows, dim)(idx_flat, activity_embedding)
    return out.reshape(t, b, dim)


# speedup vs baseline: 1.0066x; 1.0066x over previous
import functools

import jax
import jax.numpy as jnp
from jax import lax
from jax.experimental import pallas as pl
from jax.experimental.pallas import tpu as pltpu
from jax.experimental.pallas import tpu_sc as plsc

K = 8


def _make_embed(n_t: int, n_b: int, dim: int):
    info = plsc.get_sparse_core_info()
    nc, ns = info.num_cores, info.num_subcores
    nw = nc * ns
    cols = n_b // nw
    n_grp = n_t // K

    mesh = plsc.VectorSubcoreMesh(core_axis_name="c", subcore_axis_name="s")

    row_bufs = [pltpu.VMEM((cols, dim), jnp.float32) for _ in range(K)]
    gsems = [pltpu.SemaphoreType.DMA for _ in range(K)]
    wsems = [pltpu.SemaphoreType.DMA for _ in range(K)]

    @functools.partial(
        pl.kernel,
        mesh=mesh,
        out_type=jax.ShapeDtypeStruct((n_t, n_b, dim), jnp.float32),
        scratch_types=[pltpu.VMEM((n_t, cols), jnp.int32)]
        + row_bufs
        + gsems
        + wsems,
        compiler_params=pltpu.CompilerParams(use_tc_tiling_on_sc=False),
    )
    def embed(idx_hbm, table_hbm, out_hbm, idx_v, *bufs_and_sems):
        rows_v = bufs_and_sems[:K]
        gsem = bufs_and_sems[K : 2 * K]
        wsem = bufs_and_sems[2 * K : 3 * K]
        wid = lax.axis_index("s") * nc + lax.axis_index("c")
        col0 = wid * cols

        pltpu.sync_copy(idx_hbm.at[:, pl.ds(col0, cols)], idx_v)

        def gather(t_dyn, b):
            pltpu.async_copy(
                table_hbm.at[idx_v.at[t_dyn]],
                rows_v[b],
                gsem[b],
            )

        def wait_gather(b):
            pltpu.make_async_copy(
                table_hbm.at[idx_v.at[0]], rows_v[b], gsem[b]
            ).wait()

        def writeback(t_dyn, b):
            pltpu.async_copy(
                rows_v[b],
                out_hbm.at[t_dyn, pl.ds(col0, cols), :],
                wsem[b],
            )

        def wait_writeback(b):
            pltpu.make_async_copy(
                rows_v[b],
                out_hbm.at[0, pl.ds(col0, cols), :],
                wsem[b],
            ).wait()

        for b in range(K):
            gather(b, b)

        def body(g, carry):
            t0 = g * K
            for b in range(K):
                wait_gather(b)
                writeback(t0 + b, b)

            @pl.when(g + 1 < n_grp)
            def _():
                for b in range(K):
                    wait_writeback(b)
                    gather(t0 + K + b, b)

            return carry

        lax.fori_loop(0, n_grp, body, 0)
        for b in range(K):
            wait_writeback(b)

    return embed


def kernel(event_activities, activity_embedding):
    n_t, n_b = event_activities.shape
    vocab, dim = activity_embedding.shape
    return _make_embed(n_t, n_b, dim)(event_activities, activity_embedding)
